# baseline (speedup 1.0000x reference)
import functools

import jax
import jax.numpy as jnp
from jax import lax
from jax.experimental import pallas as pl
from jax.experimental.pallas import tpu as pltpu
from jax.experimental.pallas import tpu_sc as plsc

NC = 2
NS = 16
NW = NC * NS
CH = 16
QP = 4
NUM_CLASSES = 6
HID = 160
G = 128


def _sc_mesh():
  return plsc.VectorSubcoreMesh(core_axis_name="c", subcore_axis_name="s")


def _deg_count(dst2d, zeros1, np_, rows_per_tile):
  rpt = np_ // NS

  @functools.partial(
      pl.kernel,
      out_type=jax.ShapeDtypeStruct((NC * np_,), jnp.float32),
      mesh=_sc_mesh(),
      scratch_types=[
          pltpu.VMEM_SHARED((np_,), jnp.float32),
          pltpu.VMEM((CH, 128), jnp.int32),
          pltpu.VMEM((128,), jnp.float32),
          pltpu.VMEM((np_ // NS,), jnp.float32),
          pltpu.SemaphoreType.DMA,
      ],
  )
  def k(dst_hbm, z_hbm, out_hbm, deg_sh, idx_v, ones_v, stage_v, sem):
    cid = lax.axis_index("c")
    sid = lax.axis_index("s")
    r0 = sid * rpt
    pltpu.sync_copy(z_hbm.at[pl.ds(r0, rpt)], stage_v)
    pltpu.sync_copy(stage_v, deg_sh.at[pl.ds(r0, rpt)])
    for i in range(8):
      ones_v[pl.ds(i * 16, 16)] = jnp.ones((16,), jnp.float32)
    plsc.subcore_barrier()
    base = (cid * NS + sid) * rows_per_tile

    def blk(b, carry):
      pltpu.sync_copy(dst_hbm.at[pl.ds(base + b * CH, CH)], idx_v)
      descs = [
          pltpu.async_copy(ones_v, deg_sh.at[idx_v.at[j]], sem, add=True)
          for j in range(CH)
      ]
      for d in descs:
        d.wait()
      return carry

    lax.fori_loop(0, rows_per_tile // CH, blk, 0)
    plsc.subcore_barrier()
    pltpu.sync_copy(deg_sh.at[pl.ds(r0, rpt)], stage_v)
    pltpu.sync_copy(stage_v, out_hbm.at[pl.ds(cid * np_ + r0, rpt)])

  return k(dst2d, zeros1).reshape(NC, np_)


def _edge_pass(src2d, dst2d, tab_cols, zeros1, np_, w, rows_per_tile):
  rpt = np_ // NS

  @functools.partial(
      pl.kernel,
      out_type=[jax.ShapeDtypeStruct((NC * np_,), jnp.float32)
                for _ in range(w)],
      mesh=_sc_mesh(),
      compiler_params=pltpu.CompilerParams(needs_layout_passes=False),
      scratch_types=[
          pltpu.VMEM_SHARED((QP * np_,), jnp.float32),
          pltpu.VMEM((np_,), jnp.float32),
          pltpu.VMEM((np_,), jnp.float32),
          pltpu.VMEM((CH, 128), jnp.int32),
          pltpu.VMEM((CH, 128), jnp.int32),
          pltpu.VMEM((rpt,), jnp.float32),
          pltpu.VMEM((rpt,), jnp.float32),
      ],
  )
  def k(src_hbm, dst_hbm, *rest):
    tabs_hbm = rest[:w]
    z_hbm = rest[w]
    outs_hbm = rest[w + 1:2 * w + 1]
    part_sh, tab_v, agg_v, sidx, didx, red_v, tmp_v = rest[2 * w + 1:]
    cid = lax.axis_index("c")
    sid = lax.axis_index("s")
    r0 = sid * rpt
    base = (cid * NS + sid) * rows_per_tile

    for col in range(w):
      pltpu.sync_copy(z_hbm, agg_v)
      pltpu.sync_copy(tabs_hbm[col], tab_v)

      def blk(b, carry):
        pltpu.sync_copy(src_hbm.at[pl.ds(base + b * CH, CH)], sidx)
        pltpu.sync_copy(dst_hbm.at[pl.ds(base + b * CH, CH)], didx)

        def row(j, c2):
          def chunk(kk, c3):
            s16 = sidx[j, pl.ds(kk * 16, 16)]
            d16 = didx[j, pl.ds(kk * 16, 16)]
            val = plsc.load_gather(tab_v, [s16])
            plsc.addupdate_scatter(agg_v, [d16], val)
            return c3

          lax.fori_loop(0, 8, chunk, 0)
          return c2

        lax.fori_loop(0, CH, row, 0)
        return carry

      lax.fori_loop(0, rows_per_tile // CH, blk, 0)

      def addc(i, c2):
        red_v[pl.ds(i * 16, 16)] = (
            red_v[pl.ds(i * 16, 16)] + tmp_v[pl.ds(i * 16, 16)])
        return c2

      def redt(t, c):
        pltpu.sync_copy(part_sh.at[pl.ds(t * np_ + r0, rpt)], tmp_v)
        lax.fori_loop(0, rpt // 16, addc, 0)
        return c

      for ph in range(NS // QP):
        @pl.when((sid >= ph * QP) & (sid < (ph + 1) * QP))
        def _():
          pltpu.sync_copy(agg_v, part_sh.at[pl.ds((sid - ph * QP) * np_,
                                                  np_)])

        plsc.subcore_barrier()
        if ph == 0:
          pltpu.sync_copy(part_sh.at[pl.ds(r0, rpt)], red_v)
          lax.fori_loop(1, QP, redt, 0)
        else:
          lax.fori_loop(0, QP, redt, 0)
        plsc.subcore_barrier()

      pltpu.sync_copy(red_v, outs_hbm[col].at[pl.ds(cid * np_ + r0, rpt)])
      plsc.subcore_barrier()

  cols = k(src2d, dst2d, *tab_cols, zeros1)
  return jnp.concatenate(
      [c.reshape(NC, 1, np_) for c in cols], axis=1)


def _tc_scale(degp, x_t):
  np_ = degp.shape[1]
  f = x_t.shape[0]

  def body(degp_ref, xt_ref, xst_ref, dinv_ref):
    deg = degp_ref[0:1, :] + degp_ref[1:2, :] + 1.0
    dinv = lax.rsqrt(deg)
    dinv_ref[...] = dinv
    xst_ref[...] = xt_ref[...] * dinv

  return pl.pallas_call(
      body,
      out_shape=(
          jax.ShapeDtypeStruct((f, np_), jnp.float32),
          jax.ShapeDtypeStruct((1, np_), jnp.float32),
      ),
  )(degp, x_t)


def _tc_dense(aggp_t, xs_t, dinv, w1_t, b1_c, w2_t, bn=1792):
  np_ = xs_t.shape[1]
  grid = np_ // bn

  def body(aggp_ref, xst_ref, dinv_ref, w1t_ref, b1_ref, w2t_ref, out_ref):
    di = dinv_ref[...]
    agg = di * (aggp_ref[0] + aggp_ref[1] + xst_ref[...])
    h = jnp.dot(w1t_ref[...], agg, preferred_element_type=jnp.float32)
    h = jnp.maximum(h + b1_ref[...], 0.0)
    out_ref[...] = di * jnp.dot(
        w2t_ref[...], h, preferred_element_type=jnp.float32)

  return pl.pallas_call(
      body,
      grid=(grid,),
      in_specs=[
          pl.BlockSpec((NC, 4, bn), lambda i: (0, 0, i)),
          pl.BlockSpec((4, bn), lambda i: (0, i)),
          pl.BlockSpec((1, bn), lambda i: (0, i)),
          pl.BlockSpec((HID, 4), lambda i: (0, 0)),
          pl.BlockSpec((HID, 1), lambda i: (0, 0)),
          pl.BlockSpec((NUM_CLASSES, HID), lambda i: (0, 0)),
      ],
      out_specs=pl.BlockSpec((NUM_CLASSES, bn), lambda i: (0, i)),
      out_shape=jax.ShapeDtypeStruct((NUM_CLASSES, np_), jnp.float32),
  )(aggp_t, xs_t, dinv, w1_t, b1_c, w2_t)


def _tc_pool(aggp_t, hws_t, dinv, batch_p, b2_r, bn=1792):
  np_ = hws_t.shape[1]
  grid = np_ // bn
  c = NUM_CLASSES

  def body(aggp_ref, hwst_ref, dinv_ref, batch_ref, b2_ref, out_ref, acc_ref,
           cnt_ref):
    i = pl.program_id(0)

    @pl.when(i == 0)
    def _():
      acc_ref[...] = jnp.zeros_like(acc_ref)
      cnt_ref[...] = jnp.zeros_like(cnt_ref)

    agg2 = dinv_ref[...] * (aggp_ref[0] + aggp_ref[1] + hwst_ref[...])
    m = (lax.broadcasted_iota(jnp.int32, (G, bn), 0) == batch_ref[...]
         ).astype(jnp.float32)
    acc_ref[...] += lax.dot_general(
        m, agg2, (((1,), (1,)), ((), ())), preferred_element_type=jnp.float32)
    cnt_ref[...] += jnp.sum(m, axis=1, keepdims=True)

    @pl.when(i == grid - 1)
    def _():
      pooled = acc_ref[...] / jnp.maximum(cnt_ref[...], 1.0) + b2_ref[...]
      mx = jnp.max(pooled, axis=1, keepdims=True)
      ex = jnp.exp(pooled - mx)
      out_ref[...] = (pooled - mx) - jnp.log(
          jnp.sum(ex, axis=1, keepdims=True))

  return pl.pallas_call(
      body,
      grid=(grid,),
      in_specs=[
          pl.BlockSpec((NC, c, bn), lambda i: (0, 0, i)),
          pl.BlockSpec((c, bn), lambda i: (0, i)),
          pl.BlockSpec((1, bn), lambda i: (0, i)),
          pl.BlockSpec((1, bn), lambda i: (0, i)),
          pl.BlockSpec((1, c), lambda i: (0, 0)),
      ],
      out_specs=pl.BlockSpec((G, c), lambda i: (0, 0)),
      out_shape=jax.ShapeDtypeStruct((G, c), jnp.float32),
      scratch_shapes=[
          pltpu.VMEM((G, c), jnp.float32),
          pltpu.VMEM((G, 1), jnp.float32),
      ],
  )(aggp_t, hws_t, dinv, batch_p, b2_r)


def kernel(x, edge_index, batch, W1, b1, W2, b2):
  n, f = x.shape
  e = edge_index.shape[1]

  np_ = -(-(n + 1) // 256) * 256
  rows_per_tile = -(-e // (NW * 128))
  rows_per_tile = -(-rows_per_tile // CH) * CH
  ep = NW * 128 * rows_per_tile

  pad = jnp.full((ep - e,), n, jnp.int32)
  src2d = jnp.concatenate([edge_index[0], pad]).reshape(NW * rows_per_tile,
                                                        128)
  dst2d = jnp.concatenate([edge_index[1], pad]).reshape(NW * rows_per_tile,
                                                        128)

  z1 = jnp.zeros((np_,), jnp.float32)
  x_t = jnp.zeros((f, np_), jnp.float32).at[:, :n].set(x.T)
  batch_p = jnp.concatenate(
      [batch.astype(jnp.int32),
       jnp.full((np_ - n,), G, jnp.int32)]).reshape(1, np_)

  degp = _deg_count(dst2d, z1, np_, rows_per_tile)
  xs_t, dinv = _tc_scale(degp, x_t)
  agg1p = _edge_pass(src2d, dst2d, [xs_t[c] for c in range(f)], z1, np_, f,
                     rows_per_tile)
  hws_t = _tc_dense(agg1p, xs_t, dinv, W1.T, b1.reshape(-1, 1), W2.T)
  agg2p = _edge_pass(src2d, dst2d, [hws_t[c] for c in range(NUM_CLASSES)],
                     z1, np_, NUM_CLASSES, rows_per_tile)
  return _tc_pool(agg2p, hws_t, dinv, batch_p, b2.reshape(1, -1))


# speedup vs baseline: 1.0003x; 1.0003x over previous
import functools

import jax
import jax.numpy as jnp
from jax import lax
from jax.experimental import pallas as pl
from jax.experimental.pallas import tpu as pltpu
from jax.experimental.pallas import tpu_sc as plsc

NC = 2
NS = 16
NW = NC * NS
CH = 16
QP = 4
NUM_CLASSES = 6
HID = 160
G = 128


def _sc_mesh():
  return plsc.VectorSubcoreMesh(core_axis_name="c", subcore_axis_name="s")


def _deg_count(dst2d, zeros1, np_, rows_per_tile):
  rpt = np_ // NS

  @functools.partial(
      pl.kernel,
      out_type=jax.ShapeDtypeStruct((NC * np_,), jnp.float32),
      mesh=_sc_mesh(),
      scratch_types=[
          pltpu.VMEM_SHARED((np_,), jnp.float32),
          pltpu.VMEM((CH, 128), jnp.int32),
          pltpu.VMEM((128,), jnp.float32),
          pltpu.VMEM((np_ // NS,), jnp.float32),
          pltpu.SemaphoreType.DMA,
      ],
  )
  def k(dst_hbm, z_hbm, out_hbm, deg_sh, idx_v, ones_v, stage_v, sem):
    cid = lax.axis_index("c")
    sid = lax.axis_index("s")
    r0 = sid * rpt
    pltpu.sync_copy(z_hbm.at[pl.ds(r0, rpt)], stage_v)
    pltpu.sync_copy(stage_v, deg_sh.at[pl.ds(r0, rpt)])
    for i in range(8):
      ones_v[pl.ds(i * 16, 16)] = jnp.ones((16,), jnp.float32)
    plsc.subcore_barrier()
    base = (cid * NS + sid) * rows_per_tile

    def blk(b, carry):
      pltpu.sync_copy(dst_hbm.at[pl.ds(base + b * CH, CH)], idx_v)
      descs = [
          pltpu.async_copy(ones_v, deg_sh.at[idx_v.at[j]], sem, add=True)
          for j in range(CH)
      ]
      for d in descs:
        d.wait()
      return carry

    lax.fori_loop(0, rows_per_tile // CH, blk, 0)
    plsc.subcore_barrier()
    pltpu.sync_copy(deg_sh.at[pl.ds(r0, rpt)], stage_v)
    pltpu.sync_copy(stage_v, out_hbm.at[pl.ds(cid * np_ + r0, rpt)])

  return k(dst2d, zeros1).reshape(NC, np_)


def _edge_pass(src2d, dst2d, tab_cols, zeros1, np_, w, rows_per_tile):
  rpt = np_ // NS

  @functools.partial(
      pl.kernel,
      out_type=[jax.ShapeDtypeStruct((NC * np_,), jnp.float32)
                for _ in range(w)],
      mesh=_sc_mesh(),
      compiler_params=pltpu.CompilerParams(needs_layout_passes=False),
      scratch_types=[
          pltpu.VMEM_SHARED((QP * np_,), jnp.float32),
          pltpu.VMEM((np_,), jnp.float32),
          pltpu.VMEM((np_,), jnp.float32),
          pltpu.VMEM((CH, 128), jnp.int32),
          pltpu.VMEM((CH, 128), jnp.int32),
          pltpu.VMEM((rpt,), jnp.float32),
          pltpu.VMEM((rpt,), jnp.float32),
      ],
  )
  def k(src_hbm, dst_hbm, *rest):
    tabs_hbm = rest[:w]
    z_hbm = rest[w]
    outs_hbm = rest[w + 1:2 * w + 1]
    part_sh, tab_v, agg_v, sidx, didx, red_v, tmp_v = rest[2 * w + 1:]
    cid = lax.axis_index("c")
    sid = lax.axis_index("s")
    r0 = sid * rpt
    base = (cid * NS + sid) * rows_per_tile

    for col in range(w):
      pltpu.sync_copy(z_hbm, agg_v)
      pltpu.sync_copy(tabs_hbm[col], tab_v)

      def blk(b, carry):
        pltpu.sync_copy(src_hbm.at[pl.ds(base + b * CH, CH)], sidx)
        pltpu.sync_copy(dst_hbm.at[pl.ds(base + b * CH, CH)], didx)

        def row(j, c2):
          for kk in range(8):
            s16 = sidx[j, pl.ds(kk * 16, 16)]
            d16 = didx[j, pl.ds(kk * 16, 16)]
            val = plsc.load_gather(tab_v, [s16])
            plsc.addupdate_scatter(agg_v, [d16], val)
          return c2

        lax.fori_loop(0, CH, row, 0)
        return carry

      lax.fori_loop(0, rows_per_tile // CH, blk, 0)

      def addc(i, c2):
        red_v[pl.ds(i * 16, 16)] = (
            red_v[pl.ds(i * 16, 16)] + tmp_v[pl.ds(i * 16, 16)])
        return c2

      def redt(t, c):
        pltpu.sync_copy(part_sh.at[pl.ds(t * np_ + r0, rpt)], tmp_v)
        lax.fori_loop(0, rpt // 16, addc, 0)
        return c

      for ph in range(NS // QP):
        @pl.when((sid >= ph * QP) & (sid < (ph + 1) * QP))
        def _():
          pltpu.sync_copy(agg_v, part_sh.at[pl.ds((sid - ph * QP) * np_,
                                                  np_)])

        plsc.subcore_barrier()
        if ph == 0:
          pltpu.sync_copy(part_sh.at[pl.ds(r0, rpt)], red_v)
          lax.fori_loop(1, QP, redt, 0)
        else:
          lax.fori_loop(0, QP, redt, 0)
        plsc.subcore_barrier()

      pltpu.sync_copy(red_v, outs_hbm[col].at[pl.ds(cid * np_ + r0, rpt)])
      plsc.subcore_barrier()

  cols = k(src2d, dst2d, *tab_cols, zeros1)
  return jnp.concatenate(
      [c.reshape(NC, 1, np_) for c in cols], axis=1)


def _tc_scale(degp, x_t):
  np_ = degp.shape[1]
  f = x_t.shape[0]

  def body(degp_ref, xt_ref, xst_ref, dinv_ref):
    deg = degp_ref[0:1, :] + degp_ref[1:2, :] + 1.0
    dinv = lax.rsqrt(deg)
    dinv_ref[...] = dinv
    xst_ref[...] = xt_ref[...] * dinv

  return pl.pallas_call(
      body,
      out_shape=(
          jax.ShapeDtypeStruct((f, np_), jnp.float32),
          jax.ShapeDtypeStruct((1, np_), jnp.float32),
      ),
  )(degp, x_t)


def _tc_dense(aggp_t, xs_t, dinv, w1_t, b1_c, w2_t, bn=1792):
  np_ = xs_t.shape[1]
  grid = np_ // bn

  def body(aggp_ref, xst_ref, dinv_ref, w1t_ref, b1_ref, w2t_ref, out_ref):
    di = dinv_ref[...]
    agg = di * (aggp_ref[0] + aggp_ref[1] + xst_ref[...])
    h = jnp.dot(w1t_ref[...], agg, preferred_element_type=jnp.float32)
    h = jnp.maximum(h + b1_ref[...], 0.0)
    out_ref[...] = di * jnp.dot(
        w2t_ref[...], h, preferred_element_type=jnp.float32)

  return pl.pallas_call(
      body,
      grid=(grid,),
      in_specs=[
          pl.BlockSpec((NC, 4, bn), lambda i: (0, 0, i)),
          pl.BlockSpec((4, bn), lambda i: (0, i)),
          pl.BlockSpec((1, bn), lambda i: (0, i)),
          pl.BlockSpec((HID, 4), lambda i: (0, 0)),
          pl.BlockSpec((HID, 1), lambda i: (0, 0)),
          pl.BlockSpec((NUM_CLASSES, HID), lambda i: (0, 0)),
      ],
      out_specs=pl.BlockSpec((NUM_CLASSES, bn), lambda i: (0, i)),
      out_shape=jax.ShapeDtypeStruct((NUM_CLASSES, np_), jnp.float32),
  )(aggp_t, xs_t, dinv, w1_t, b1_c, w2_t)


def _tc_pool(aggp_t, hws_t, dinv, batch_p, b2_r, bn=1792):
  np_ = hws_t.shape[1]
  grid = np_ // bn
  c = NUM_CLASSES

  def body(aggp_ref, hwst_ref, dinv_ref, batch_ref, b2_ref, out_ref, acc_ref,
           cnt_ref):
    i = pl.program_id(0)

    @pl.when(i == 0)
    def _():
      acc_ref[...] = jnp.zeros_like(acc_ref)
      cnt_ref[...] = jnp.zeros_like(cnt_ref)

    agg2 = dinv_ref[...] * (aggp_ref[0] + aggp_ref[1] + hwst_ref[...])
    m = (lax.broadcasted_iota(jnp.int32, (G, bn), 0) == batch_ref[...]
         ).astype(jnp.float32)
    acc_ref[...] += lax.dot_general(
        m, agg2, (((1,), (1,)), ((), ())), preferred_element_type=jnp.float32)
    cnt_ref[...] += jnp.sum(m, axis=1, keepdims=True)

    @pl.when(i == grid - 1)
    def _():
      pooled = acc_ref[...] / jnp.maximum(cnt_ref[...], 1.0) + b2_ref[...]
      mx = jnp.max(pooled, axis=1, keepdims=True)
      ex = jnp.exp(pooled - mx)
      out_ref[...] = (pooled - mx) - jnp.log(
          jnp.sum(ex, axis=1, keepdims=True))

  return pl.pallas_call(
      body,
      grid=(grid,),
      in_specs=[
          pl.BlockSpec((NC, c, bn), lambda i: (0, 0, i)),
          pl.BlockSpec((c, bn), lambda i: (0, i)),
          pl.BlockSpec((1, bn), lambda i: (0, i)),
          pl.BlockSpec((1, bn), lambda i: (0, i)),
          pl.BlockSpec((1, c), lambda i: (0, 0)),
      ],
      out_specs=pl.BlockSpec((G, c), lambda i: (0, 0)),
      out_shape=jax.ShapeDtypeStruct((G, c), jnp.float32),
      scratch_shapes=[
          pltpu.VMEM((G, c), jnp.float32),
          pltpu.VMEM((G, 1), jnp.float32),
      ],
  )(aggp_t, hws_t, dinv, batch_p, b2_r)


def kernel(x, edge_index, batch, W1, b1, W2, b2):
  n, f = x.shape
  e = edge_index.shape[1]

  np_ = -(-(n + 1) // 256) * 256
  rows_per_tile = -(-e // (NW * 128))
  rows_per_tile = -(-rows_per_tile // CH) * CH
  ep = NW * 128 * rows_per_tile

  pad = jnp.full((ep - e,), n, jnp.int32)
  src2d = jnp.concatenate([edge_index[0], pad]).reshape(NW * rows_per_tile,
                                                        128)
  dst2d = jnp.concatenate([edge_index[1], pad]).reshape(NW * rows_per_tile,
                                                        128)

  z1 = jnp.zeros((np_,), jnp.float32)
  x_t = jnp.zeros((f, np_), jnp.float32).at[:, :n].set(x.T)
  batch_p = jnp.concatenate(
      [batch.astype(jnp.int32),
       jnp.full((np_ - n,), G, jnp.int32)]).reshape(1, np_)

  degp = _deg_count(dst2d, z1, np_, rows_per_tile)
  xs_t, dinv = _tc_scale(degp, x_t)
  agg1p = _edge_pass(src2d, dst2d, [xs_t[c] for c in range(f)], z1, np_, f,
                     rows_per_tile)
  hws_t = _tc_dense(agg1p, xs_t, dinv, W1.T, b1.reshape(-1, 1), W2.T)
  agg2p = _edge_pass(src2d, dst2d, [hws_t[c] for c in range(NUM_CLASSES)],
                     z1, np_, NUM_CLASSES, rows_per_tile)
  return _tc_pool(agg2p, hws_t, dinv, batch_p, b2.reshape(1, -1))


# speedup vs baseline: 1.2554x; 1.2550x over previous
import functools

import jax
import jax.numpy as jnp
from jax import lax
from jax.experimental import pallas as pl
from jax.experimental.pallas import tpu as pltpu
from jax.experimental.pallas import tpu_sc as plsc

NC = 2
NS = 16
NW = NC * NS
CH = 16
NUM_CLASSES = 6
HID = 160
G = 128


def _sc_mesh():
  return plsc.VectorSubcoreMesh(core_axis_name="c", subcore_axis_name="s")


def _deg_count(dst2d, zeros1, np_, rows0, rows1):
  rpt = np_ // NS

  @functools.partial(
      pl.kernel,
      out_type=jax.ShapeDtypeStruct((NC * np_,), jnp.float32),
      mesh=_sc_mesh(),
      scratch_types=[
          pltpu.VMEM_SHARED((np_,), jnp.float32),
          pltpu.VMEM((CH, 128), jnp.int32),
          pltpu.VMEM((128,), jnp.float32),
          pltpu.VMEM((np_ // NS,), jnp.float32),
          pltpu.SemaphoreType.DMA,
      ],
  )
  def k(dst_hbm, z_hbm, out_hbm, deg_sh, idx_v, ones_v, stage_v, sem):
    cid = lax.axis_index("c")
    sid = lax.axis_index("s")
    r0 = sid * rpt
    pltpu.sync_copy(z_hbm.at[pl.ds(r0, rpt)], stage_v)
    pltpu.sync_copy(stage_v, deg_sh.at[pl.ds(r0, rpt)])
    for i in range(8):
      ones_v[pl.ds(i * 16, 16)] = jnp.ones((16,), jnp.float32)
    plsc.subcore_barrier()
    rows_this = jnp.where(cid == 0, rows0, rows1)
    base = cid * NS * rows0 + sid * rows_this

    def blk(b, carry):
      pltpu.sync_copy(dst_hbm.at[pl.ds(base + b * CH, CH)], idx_v)
      descs = [
          pltpu.async_copy(ones_v, deg_sh.at[idx_v.at[j]], sem, add=True)
          for j in range(CH)
      ]
      for d in descs:
        d.wait()
      return carry

    lax.fori_loop(0, rows_this // CH, blk, 0)
    plsc.subcore_barrier()
    pltpu.sync_copy(deg_sh.at[pl.ds(r0, rpt)], stage_v)
    pltpu.sync_copy(stage_v, out_hbm.at[pl.ds(cid * np_ + r0, rpt)])

  return k(dst2d, zeros1).reshape(NC, np_)


def _edge_pass(src2d, dst2d, tab_cols, zeros1, np_, w, rows0, rows1):
  rpt = np_ // NS

  @functools.partial(
      pl.kernel,
      out_type=[jax.ShapeDtypeStruct((NC * np_,), jnp.float32)
                for _ in range(w)],
      mesh=_sc_mesh(),
      scratch_types=(
          [pltpu.VMEM_SHARED((np_,), jnp.float32) for _ in range(w)] +
          [pltpu.VMEM_SHARED((np_,), jnp.float32) for _ in range(w)] + [
              pltpu.VMEM((CH, 128), jnp.int32),
              pltpu.VMEM((CH, 128), jnp.int32),
              pltpu.VMEM((w, 128), jnp.float32),
              pltpu.VMEM((w, 128), jnp.float32),
              pltpu.VMEM((np_ // NS,), jnp.float32),
              pltpu.SemaphoreType.DMA,
              pltpu.SemaphoreType.DMA,
          ]),
  )
  def k(src_hbm, dst_hbm, *rest):
    tabs_hbm = rest[:w]
    z_hbm = rest[w]
    outs_hbm = rest[w + 1:2 * w + 1]
    tab_sh = rest[2 * w + 1:3 * w + 1]
    agg_sh = rest[3 * w + 1:4 * w + 1]
    sidx, didx, bufa, bufb, stage_v, semg, sems = rest[4 * w + 1:]
    cid = lax.axis_index("c")
    sid = lax.axis_index("s")
    r0 = sid * rpt
    pltpu.sync_copy(z_hbm.at[pl.ds(r0, rpt)], stage_v)
    for col in range(w):
      pltpu.sync_copy(stage_v, agg_sh[col].at[pl.ds(r0, rpt)])
    for col in range(w):
      pltpu.sync_copy(tabs_hbm[col].at[pl.ds(r0, rpt)], stage_v)
      pltpu.sync_copy(stage_v, tab_sh[col].at[pl.ds(r0, rpt)])
    plsc.subcore_barrier()
    rows_this = jnp.where(cid == 0, rows0, rows1)
    base = cid * NS * rows0 + sid * rows_this

    def blk(b, carry):
      pltpu.sync_copy(src_hbm.at[pl.ds(base + b * CH, CH)], sidx)
      pltpu.sync_copy(dst_hbm.at[pl.ds(base + b * CH, CH)], didx)

      def rowpair(p, c2):
        j0 = 2 * p
        j1 = j0 + 1
        ga = [
            pltpu.async_copy(tab_sh[col].at[sidx.at[j0]], bufa.at[col], semg)
            for col in range(w)
        ]
        gb = [
            pltpu.async_copy(tab_sh[col].at[sidx.at[j1]], bufb.at[col], semg)
            for col in range(w)
        ]
        for d in ga:
          d.wait()
        sa = [
            pltpu.async_copy(bufa.at[col], agg_sh[col].at[didx.at[j0]], sems,
                             add=True) for col in range(w)
        ]
        for d in gb:
          d.wait()
        sb = [
            pltpu.async_copy(bufb.at[col], agg_sh[col].at[didx.at[j1]], sems,
                             add=True) for col in range(w)
        ]
        for d in sa + sb:
          d.wait()
        return c2

      lax.fori_loop(0, CH // 2, rowpair, 0)
      return carry

    lax.fori_loop(0, rows_this // CH, blk, 0)
    plsc.subcore_barrier()
    for col in range(w):
      pltpu.sync_copy(agg_sh[col].at[pl.ds(r0, rpt)], stage_v)
      pltpu.sync_copy(stage_v, outs_hbm[col].at[pl.ds(cid * np_ + r0, rpt)])

  cols = k(src2d, dst2d, *tab_cols, zeros1)
  return jnp.concatenate(
      [c.reshape(NC, 1, np_) for c in cols], axis=1)


def _tc_scale(degp, x_t):
  np_ = degp.shape[1]
  f = x_t.shape[0]

  def body(degp_ref, xt_ref, xst_ref, dinv_ref):
    deg = degp_ref[0:1, :] + degp_ref[1:2, :] + 1.0
    dinv = lax.rsqrt(deg)
    dinv_ref[...] = dinv
    xst_ref[...] = xt_ref[...] * dinv

  return pl.pallas_call(
      body,
      out_shape=(
          jax.ShapeDtypeStruct((f, np_), jnp.float32),
          jax.ShapeDtypeStruct((1, np_), jnp.float32),
      ),
  )(degp, x_t)


def _tc_dense(aggp_t, xs_t, dinv, w1_t, b1_c, w2_t, bn=1792):
  np_ = xs_t.shape[1]
  grid = np_ // bn

  def body(aggp_ref, xst_ref, dinv_ref, w1t_ref, b1_ref, w2t_ref, out_ref):
    di = dinv_ref[...]
    agg = di * (aggp_ref[0] + aggp_ref[1] + xst_ref[...])
    h = jnp.dot(w1t_ref[...], agg, preferred_element_type=jnp.float32)
    h = jnp.maximum(h + b1_ref[...], 0.0)
    out_ref[...] = di * jnp.dot(
        w2t_ref[...], h, preferred_element_type=jnp.float32)

  return pl.pallas_call(
      body,
      grid=(grid,),
      in_specs=[
          pl.BlockSpec((NC, 4, bn), lambda i: (0, 0, i)),
          pl.BlockSpec((4, bn), lambda i: (0, i)),
          pl.BlockSpec((1, bn), lambda i: (0, i)),
          pl.BlockSpec((HID, 4), lambda i: (0, 0)),
          pl.BlockSpec((HID, 1), lambda i: (0, 0)),
          pl.BlockSpec((NUM_CLASSES, HID), lambda i: (0, 0)),
      ],
      out_specs=pl.BlockSpec((NUM_CLASSES, bn), lambda i: (0, i)),
      out_shape=jax.ShapeDtypeStruct((NUM_CLASSES, np_), jnp.float32),
  )(aggp_t, xs_t, dinv, w1_t, b1_c, w2_t)


def _tc_pool(aggp_t, hws_t, dinv, batch_p, b2_r, bn=1792):
  np_ = hws_t.shape[1]
  grid = np_ // bn
  c = NUM_CLASSES

  def body(aggp_ref, hwst_ref, dinv_ref, batch_ref, b2_ref, out_ref, acc_ref,
           cnt_ref):
    i = pl.program_id(0)

    @pl.when(i == 0)
    def _():
      acc_ref[...] = jnp.zeros_like(acc_ref)
      cnt_ref[...] = jnp.zeros_like(cnt_ref)

    agg2 = dinv_ref[...] * (aggp_ref[0] + aggp_ref[1] + hwst_ref[...])
    m = (lax.broadcasted_iota(jnp.int32, (G, bn), 0) == batch_ref[...]
         ).astype(jnp.float32)
    acc_ref[...] += lax.dot_general(
        m, agg2, (((1,), (1,)), ((), ())), preferred_element_type=jnp.float32)
    cnt_ref[...] += jnp.sum(m, axis=1, keepdims=True)

    @pl.when(i == grid - 1)
    def _():
      pooled = acc_ref[...] / jnp.maximum(cnt_ref[...], 1.0) + b2_ref[...]
      mx = jnp.max(pooled, axis=1, keepdims=True)
      ex = jnp.exp(pooled - mx)
      out_ref[...] = (pooled - mx) - jnp.log(
          jnp.sum(ex, axis=1, keepdims=True))

  return pl.pallas_call(
      body,
      grid=(grid,),
      in_specs=[
          pl.BlockSpec((NC, c, bn), lambda i: (0, 0, i)),
          pl.BlockSpec((c, bn), lambda i: (0, i)),
          pl.BlockSpec((1, bn), lambda i: (0, i)),
          pl.BlockSpec((1, bn), lambda i: (0, i)),
          pl.BlockSpec((1, c), lambda i: (0, 0)),
      ],
      out_specs=pl.BlockSpec((G, c), lambda i: (0, 0)),
      out_shape=jax.ShapeDtypeStruct((G, c), jnp.float32),
      scratch_shapes=[
          pltpu.VMEM((G, c), jnp.float32),
          pltpu.VMEM((G, 1), jnp.float32),
      ],
  )(aggp_t, hws_t, dinv, batch_p, b2_r)


def kernel(x, edge_index, batch, W1, b1, W2, b2):
  n, f = x.shape
  e = edge_index.shape[1]

  np_ = -(-(n + 1) // 256) * 256
  rows_per_tile = -(-e // (NW * 128))
  rows_per_tile = -(-rows_per_tile // CH) * CH
  rows0 = (2 * rows_per_tile * 3 // 5) // CH * CH
  rows1 = 2 * rows_per_tile - rows0
  ep = NS * 128 * (rows0 + rows1)

  pad = jnp.full((ep - e,), n, jnp.int32)
  src2d = jnp.concatenate([edge_index[0], pad]).reshape(ep // 128, 128)
  dst2d = jnp.concatenate([edge_index[1], pad]).reshape(ep // 128, 128)

  z1 = jnp.zeros((np_,), jnp.float32)
  x_t = jnp.zeros((f, np_), jnp.float32).at[:, :n].set(x.T)
  batch_p = jnp.concatenate(
      [batch.astype(jnp.int32),
       jnp.full((np_ - n,), G, jnp.int32)]).reshape(1, np_)

  degp = _deg_count(dst2d, z1, np_, rows0, rows1)
  xs_t, dinv = _tc_scale(degp, x_t)
  agg1p = _edge_pass(src2d, dst2d, [xs_t[c] for c in range(f)], z1, np_, f,
                     rows0, rows1)
  hws_t = _tc_dense(agg1p, xs_t, dinv, W1.T, b1.reshape(-1, 1), W2.T)
  agg2p = _edge_pass(src2d, dst2d, [hws_t[c] for c in range(NUM_CLASSES)],
                     z1, np_, NUM_CLASSES, rows0, rows1)
  return _tc_pool(agg2p, hws_t, dinv, batch_p, b2.reshape(1, -1))


# speedup vs baseline: 1.2855x; 1.0240x over previous
import functools

import jax
import jax.numpy as jnp
from jax import lax
from jax.experimental import pallas as pl
from jax.experimental.pallas import tpu as pltpu
from jax.experimental.pallas import tpu_sc as plsc

NC = 2
NS = 16
NW = NC * NS
CH = 16
NUM_CLASSES = 6
HID = 160
G = 128


def _sc_mesh():
  return plsc.VectorSubcoreMesh(core_axis_name="c", subcore_axis_name="s")


def _deg_count(dst2d, zeros1, np_, rows0, rows1):
  rpt = np_ // NS

  @functools.partial(
      pl.kernel,
      out_type=jax.ShapeDtypeStruct((NC * np_,), jnp.float32),
      mesh=_sc_mesh(),
      scratch_types=[
          pltpu.VMEM_SHARED((np_,), jnp.float32),
          pltpu.VMEM((CH, 128), jnp.int32),
          pltpu.VMEM((128,), jnp.float32),
          pltpu.VMEM((np_ // NS,), jnp.float32),
          pltpu.SemaphoreType.DMA,
      ],
  )
  def k(dst_hbm, z_hbm, out_hbm, deg_sh, idx_v, ones_v, stage_v, sem):
    cid = lax.axis_index("c")
    sid = lax.axis_index("s")
    r0 = sid * rpt
    pltpu.sync_copy(z_hbm.at[pl.ds(r0, rpt)], stage_v)
    pltpu.sync_copy(stage_v, deg_sh.at[pl.ds(r0, rpt)])
    for i in range(8):
      ones_v[pl.ds(i * 16, 16)] = jnp.ones((16,), jnp.float32)
    plsc.subcore_barrier()
    rows_this = jnp.where(cid == 0, rows0, rows1)
    base = cid * NS * rows0 + sid * rows_this

    def blk(b, carry):
      pltpu.sync_copy(dst_hbm.at[pl.ds(base + b * CH, CH)], idx_v)
      descs = [
          pltpu.async_copy(ones_v, deg_sh.at[idx_v.at[j]], sem, add=True)
          for j in range(CH)
      ]
      for d in descs:
        d.wait()
      return carry

    lax.fori_loop(0, rows_this // CH, blk, 0)
    plsc.subcore_barrier()
    pltpu.sync_copy(deg_sh.at[pl.ds(r0, rpt)], stage_v)
    pltpu.sync_copy(stage_v, out_hbm.at[pl.ds(cid * np_ + r0, rpt)])

  return k(dst2d, zeros1).reshape(NC, np_)


def _edge_pass(src2d, dst2d, tab_cols, zeros1, np_, w, rows0, rows1):
  rpt = np_ // NS

  @functools.partial(
      pl.kernel,
      out_type=[jax.ShapeDtypeStruct((NC * np_,), jnp.float32)
                for _ in range(w)],
      mesh=_sc_mesh(),
      scratch_types=(
          [pltpu.VMEM_SHARED((np_,), jnp.float32) for _ in range(w)] +
          [pltpu.VMEM_SHARED((np_,), jnp.float32) for _ in range(w)] + [
              pltpu.VMEM((CH, 128), jnp.int32),
              pltpu.VMEM((CH, 128), jnp.int32),
              pltpu.VMEM((w, 128), jnp.float32),
              pltpu.VMEM((w, 128), jnp.float32),
              pltpu.VMEM((np_ // NS,), jnp.float32),
              pltpu.SemaphoreType.DMA,
              pltpu.SemaphoreType.DMA,
          ]),
  )
  def k(src_hbm, dst_hbm, *rest):
    tabs_hbm = rest[:w]
    z_hbm = rest[w]
    outs_hbm = rest[w + 1:2 * w + 1]
    tab_sh = rest[2 * w + 1:3 * w + 1]
    agg_sh = rest[3 * w + 1:4 * w + 1]
    sidx, didx, bufa, bufb, stage_v, semg, sems = rest[4 * w + 1:]
    cid = lax.axis_index("c")
    sid = lax.axis_index("s")
    r0 = sid * rpt
    pltpu.sync_copy(z_hbm.at[pl.ds(r0, rpt)], stage_v)
    for col in range(w):
      pltpu.sync_copy(stage_v, agg_sh[col].at[pl.ds(r0, rpt)])
    for col in range(w):
      pltpu.sync_copy(tabs_hbm[col].at[pl.ds(r0, rpt)], stage_v)
      pltpu.sync_copy(stage_v, tab_sh[col].at[pl.ds(r0, rpt)])
    plsc.subcore_barrier()
    rows_this = jnp.where(cid == 0, rows0, rows1)
    base = cid * NS * rows0 + sid * rows_this

    def blk(b, carry):
      pltpu.sync_copy(src_hbm.at[pl.ds(base + b * CH, CH)], sidx)
      pltpu.sync_copy(dst_hbm.at[pl.ds(base + b * CH, CH)], didx)

      def rowpair(p, c2):
        j0 = 2 * p
        j1 = j0 + 1
        ga = [
            pltpu.async_copy(tab_sh[col].at[sidx.at[j0]], bufa.at[col], semg)
            for col in range(w)
        ]
        gb = [
            pltpu.async_copy(tab_sh[col].at[sidx.at[j1]], bufb.at[col], semg)
            for col in range(w)
        ]
        for d in ga:
          d.wait()
        sa = [
            pltpu.async_copy(bufa.at[col], agg_sh[col].at[didx.at[j0]], sems,
                             add=True) for col in range(w)
        ]
        for d in gb:
          d.wait()
        sb = [
            pltpu.async_copy(bufb.at[col], agg_sh[col].at[didx.at[j1]], sems,
                             add=True) for col in range(w)
        ]
        for d in sa + sb:
          d.wait()
        return c2

      lax.fori_loop(0, CH // 2, rowpair, 0)
      return carry

    lax.fori_loop(0, rows_this // CH, blk, 0)
    plsc.subcore_barrier()
    for col in range(w):
      pltpu.sync_copy(agg_sh[col].at[pl.ds(r0, rpt)], stage_v)
      pltpu.sync_copy(stage_v, outs_hbm[col].at[pl.ds(cid * np_ + r0, rpt)])

  cols = k(src2d, dst2d, *tab_cols, zeros1)
  return jnp.concatenate(
      [c.reshape(NC, 1, np_) for c in cols], axis=1)


def _tc_scale(degp, x_t):
  np_ = degp.shape[1]
  f = x_t.shape[0]

  def body(degp_ref, xt_ref, xst_ref, dinv_ref):
    deg = degp_ref[0:1, :] + degp_ref[1:2, :] + 1.0
    dinv = lax.rsqrt(deg)
    dinv_ref[...] = dinv
    xst_ref[...] = xt_ref[...] * dinv

  return pl.pallas_call(
      body,
      out_shape=(
          jax.ShapeDtypeStruct((f, np_), jnp.float32),
          jax.ShapeDtypeStruct((1, np_), jnp.float32),
      ),
  )(degp, x_t)


def _tc_dense(aggp_t, xs_t, dinv, w1_t, b1_c, w2_t, bn=1792):
  np_ = xs_t.shape[1]
  grid = np_ // bn

  def body(aggp_ref, xst_ref, dinv_ref, w1t_ref, b1_ref, w2t_ref, out_ref):
    di = dinv_ref[...]
    agg = di * (aggp_ref[0] + aggp_ref[1] + xst_ref[...])
    h = jnp.dot(w1t_ref[...], agg, preferred_element_type=jnp.float32)
    h = jnp.maximum(h + b1_ref[...], 0.0)
    out_ref[...] = di * jnp.dot(
        w2t_ref[...], h, preferred_element_type=jnp.float32)

  return pl.pallas_call(
      body,
      grid=(grid,),
      in_specs=[
          pl.BlockSpec((NC, 4, bn), lambda i: (0, 0, i)),
          pl.BlockSpec((4, bn), lambda i: (0, i)),
          pl.BlockSpec((1, bn), lambda i: (0, i)),
          pl.BlockSpec((HID, 4), lambda i: (0, 0)),
          pl.BlockSpec((HID, 1), lambda i: (0, 0)),
          pl.BlockSpec((NUM_CLASSES, HID), lambda i: (0, 0)),
      ],
      out_specs=pl.BlockSpec((NUM_CLASSES, bn), lambda i: (0, i)),
      out_shape=jax.ShapeDtypeStruct((NUM_CLASSES, np_), jnp.float32),
  )(aggp_t, xs_t, dinv, w1_t, b1_c, w2_t)


def _tc_pool(aggp_t, hws_t, dinv, batch_p, b2_r, bn=1792):
  np_ = hws_t.shape[1]
  grid = np_ // bn
  c = NUM_CLASSES

  def body(aggp_ref, hwst_ref, dinv_ref, batch_ref, b2_ref, out_ref, acc_ref,
           cnt_ref):
    i = pl.program_id(0)

    @pl.when(i == 0)
    def _():
      acc_ref[...] = jnp.zeros_like(acc_ref)
      cnt_ref[...] = jnp.zeros_like(cnt_ref)

    agg2 = dinv_ref[...] * (aggp_ref[0] + aggp_ref[1] + hwst_ref[...])
    m = (lax.broadcasted_iota(jnp.int32, (G, bn), 0) == batch_ref[...]
         ).astype(jnp.float32)
    acc_ref[...] += lax.dot_general(
        m, agg2, (((1,), (1,)), ((), ())), preferred_element_type=jnp.float32)
    cnt_ref[...] += jnp.sum(m, axis=1, keepdims=True)

    @pl.when(i == grid - 1)
    def _():
      pooled = acc_ref[...] / jnp.maximum(cnt_ref[...], 1.0) + b2_ref[...]
      mx = jnp.max(pooled, axis=1, keepdims=True)
      ex = jnp.exp(pooled - mx)
      out_ref[...] = (pooled - mx) - jnp.log(
          jnp.sum(ex, axis=1, keepdims=True))

  return pl.pallas_call(
      body,
      grid=(grid,),
      in_specs=[
          pl.BlockSpec((NC, c, bn), lambda i: (0, 0, i)),
          pl.BlockSpec((c, bn), lambda i: (0, i)),
          pl.BlockSpec((1, bn), lambda i: (0, i)),
          pl.BlockSpec((1, bn), lambda i: (0, i)),
          pl.BlockSpec((1, c), lambda i: (0, 0)),
      ],
      out_specs=pl.BlockSpec((G, c), lambda i: (0, 0)),
      out_shape=jax.ShapeDtypeStruct((G, c), jnp.float32),
      scratch_shapes=[
          pltpu.VMEM((G, c), jnp.float32),
          pltpu.VMEM((G, 1), jnp.float32),
      ],
  )(aggp_t, hws_t, dinv, batch_p, b2_r)


def kernel(x, edge_index, batch, W1, b1, W2, b2):
  n, f = x.shape
  e = edge_index.shape[1]

  np_ = -(-(n + 1) // 256) * 256
  rows_per_tile = -(-e // (NW * 128))
  rows_per_tile = -(-rows_per_tile // CH) * CH
  rows0 = (2 * rows_per_tile * 8 // 13) // CH * CH
  rows1 = 2 * rows_per_tile - rows0
  ep = NS * 128 * (rows0 + rows1)

  pad = jnp.full((ep - e,), n, jnp.int32)
  src2d = jnp.concatenate([edge_index[0], pad]).reshape(ep // 128, 128)
  dst2d = jnp.concatenate([edge_index[1], pad]).reshape(ep // 128, 128)

  z1 = jnp.zeros((np_,), jnp.float32)
  x_t = jnp.zeros((f, np_), jnp.float32).at[:, :n].set(x.T)
  batch_p = jnp.concatenate(
      [batch.astype(jnp.int32),
       jnp.full((np_ - n,), G, jnp.int32)]).reshape(1, np_)

  degp = _deg_count(dst2d, z1, np_, rows0, rows1)
  xs_t, dinv = _tc_scale(degp, x_t)
  agg1p = _edge_pass(src2d, dst2d, [xs_t[c] for c in range(f)], z1, np_, f,
                     rows0, rows1)
  hws_t = _tc_dense(agg1p, xs_t, dinv, W1.T, b1.reshape(-1, 1), W2.T)
  agg2p = _edge_pass(src2d, dst2d, [hws_t[c] for c in range(NUM_CLASSES)],
                     z1, np_, NUM_CLASSES, rows0, rows1)
  return _tc_pool(agg2p, hws_t, dinv, batch_p, b2.reshape(1, -1))


# speedup vs baseline: 1.3172x; 1.0247x over previous
import functools

import jax
import jax.numpy as jnp
from jax import lax
from jax.experimental import pallas as pl
from jax.experimental.pallas import tpu as pltpu
from jax.experimental.pallas import tpu_sc as plsc

NC = 2
NS = 16
NW = NC * NS
CH = 16
NUM_CLASSES = 6
HID = 160
G = 128


def _sc_mesh():
  return plsc.VectorSubcoreMesh(core_axis_name="c", subcore_axis_name="s")


def _deg_count(dst2d, zeros1, np_, rows0, rows1):
  rpt = np_ // NS

  @functools.partial(
      pl.kernel,
      out_type=jax.ShapeDtypeStruct((NC * np_,), jnp.float32),
      mesh=_sc_mesh(),
      scratch_types=[
          pltpu.VMEM_SHARED((np_,), jnp.float32),
          pltpu.VMEM((CH, 128), jnp.int32),
          pltpu.VMEM((128,), jnp.float32),
          pltpu.VMEM((np_ // NS,), jnp.float32),
          pltpu.SemaphoreType.DMA,
      ],
  )
  def k(dst_hbm, z_hbm, out_hbm, deg_sh, idx_v, ones_v, stage_v, sem):
    cid = lax.axis_index("c")
    sid = lax.axis_index("s")
    r0 = sid * rpt
    pltpu.sync_copy(z_hbm.at[pl.ds(r0, rpt)], stage_v)
    pltpu.sync_copy(stage_v, deg_sh.at[pl.ds(r0, rpt)])
    for i in range(8):
      ones_v[pl.ds(i * 16, 16)] = jnp.ones((16,), jnp.float32)
    plsc.subcore_barrier()
    rows_this = jnp.where(cid == 0, rows0, rows1)
    base = cid * NS * rows0 + sid * rows_this

    def blk(b, carry):
      pltpu.sync_copy(dst_hbm.at[pl.ds(base + b * CH, CH)], idx_v)
      descs = [
          pltpu.async_copy(ones_v, deg_sh.at[idx_v.at[j]], sem, add=True)
          for j in range(CH)
      ]
      for d in descs:
        d.wait()
      return carry

    lax.fori_loop(0, rows_this // CH, blk, 0)
    plsc.subcore_barrier()
    pltpu.sync_copy(deg_sh.at[pl.ds(r0, rpt)], stage_v)
    pltpu.sync_copy(stage_v, out_hbm.at[pl.ds(cid * np_ + r0, rpt)])

  return k(dst2d, zeros1).reshape(NC, np_)


def _edge_pass(src2d, dst2d, tab_cols, zeros1, np_, w, rows0, rows1):
  rpt = np_ // NS

  @functools.partial(
      pl.kernel,
      out_type=[jax.ShapeDtypeStruct((NC * np_,), jnp.float32)
                for _ in range(w)],
      mesh=_sc_mesh(),
      scratch_types=(
          [pltpu.VMEM_SHARED((np_,), jnp.float32) for _ in range(w)] +
          [pltpu.VMEM_SHARED((np_,), jnp.float32) for _ in range(w)] + [
              pltpu.VMEM((CH, 128), jnp.int32),
              pltpu.VMEM((CH, 128), jnp.int32),
              pltpu.VMEM((w, 128), jnp.float32),
              pltpu.VMEM((w, 128), jnp.float32),
              pltpu.VMEM((np_ // NS,), jnp.float32),
              pltpu.SemaphoreType.DMA,
              pltpu.SemaphoreType.DMA,
          ]),
  )
  def k(src_hbm, dst_hbm, *rest):
    tabs_hbm = rest[:w]
    z_hbm = rest[w]
    outs_hbm = rest[w + 1:2 * w + 1]
    tab_sh = rest[2 * w + 1:3 * w + 1]
    agg_sh = rest[3 * w + 1:4 * w + 1]
    sidx, didx, bufa, bufb, stage_v, semg, sems = rest[4 * w + 1:]
    cid = lax.axis_index("c")
    sid = lax.axis_index("s")
    r0 = sid * rpt
    pltpu.sync_copy(z_hbm.at[pl.ds(r0, rpt)], stage_v)
    for col in range(w):
      pltpu.sync_copy(stage_v, agg_sh[col].at[pl.ds(r0, rpt)])
    for col in range(w):
      pltpu.sync_copy(tabs_hbm[col].at[pl.ds(r0, rpt)], stage_v)
      pltpu.sync_copy(stage_v, tab_sh[col].at[pl.ds(r0, rpt)])
    plsc.subcore_barrier()
    rows_this = jnp.where(cid == 0, rows0, rows1)
    base = cid * NS * rows0 + sid * rows_this

    def blk(b, carry):
      pltpu.sync_copy(src_hbm.at[pl.ds(base + b * CH, CH)], sidx)
      pltpu.sync_copy(dst_hbm.at[pl.ds(base + b * CH, CH)], didx)

      def rowpair(p, c2):
        j0 = 2 * p
        j1 = j0 + 1
        ga = [
            pltpu.async_copy(tab_sh[col].at[sidx.at[j0]], bufa.at[col], semg)
            for col in range(w)
        ]
        gb = [
            pltpu.async_copy(tab_sh[col].at[sidx.at[j1]], bufb.at[col], semg)
            for col in range(w)
        ]
        for d in ga:
          d.wait()
        sa = [
            pltpu.async_copy(bufa.at[col], agg_sh[col].at[didx.at[j0]], sems,
                             add=True) for col in range(w)
        ]
        for d in gb:
          d.wait()
        sb = [
            pltpu.async_copy(bufb.at[col], agg_sh[col].at[didx.at[j1]], sems,
                             add=True) for col in range(w)
        ]
        for d in sa + sb:
          d.wait()
        return c2

      lax.fori_loop(0, CH // 2, rowpair, 0)
      return carry

    lax.fori_loop(0, rows_this // CH, blk, 0)
    plsc.subcore_barrier()
    for col in range(w):
      pltpu.sync_copy(agg_sh[col].at[pl.ds(r0, rpt)], stage_v)
      pltpu.sync_copy(stage_v, outs_hbm[col].at[pl.ds(cid * np_ + r0, rpt)])

  cols = k(src2d, dst2d, *tab_cols, zeros1)
  return jnp.concatenate(
      [c.reshape(NC, 1, np_) for c in cols], axis=1)


def _tc_scale(degp, x_t):
  np_ = degp.shape[1]
  f = x_t.shape[0]

  def body(degp_ref, xt_ref, xst_ref, dinv_ref):
    deg = degp_ref[0:1, :] + degp_ref[1:2, :] + 1.0
    dinv = lax.rsqrt(deg)
    dinv_ref[...] = dinv
    xst_ref[...] = xt_ref[...] * dinv

  return pl.pallas_call(
      body,
      out_shape=(
          jax.ShapeDtypeStruct((f, np_), jnp.float32),
          jax.ShapeDtypeStruct((1, np_), jnp.float32),
      ),
  )(degp, x_t)


def _tc_dense(aggp_t, xs_t, dinv, w1_t, b1_c, w2_t, bn=1792):
  np_ = xs_t.shape[1]
  grid = np_ // bn

  def body(aggp_ref, xst_ref, dinv_ref, w1t_ref, b1_ref, w2t_ref, out_ref):
    di = dinv_ref[...]
    agg = di * (aggp_ref[0] + aggp_ref[1] + xst_ref[...])
    h = jnp.dot(w1t_ref[...], agg, preferred_element_type=jnp.float32)
    h = jnp.maximum(h + b1_ref[...], 0.0)
    out_ref[...] = di * jnp.dot(
        w2t_ref[...], h, preferred_element_type=jnp.float32)

  return pl.pallas_call(
      body,
      grid=(grid,),
      in_specs=[
          pl.BlockSpec((NC, 4, bn), lambda i: (0, 0, i)),
          pl.BlockSpec((4, bn), lambda i: (0, i)),
          pl.BlockSpec((1, bn), lambda i: (0, i)),
          pl.BlockSpec((HID, 4), lambda i: (0, 0)),
          pl.BlockSpec((HID, 1), lambda i: (0, 0)),
          pl.BlockSpec((NUM_CLASSES, HID), lambda i: (0, 0)),
      ],
      out_specs=pl.BlockSpec((NUM_CLASSES, bn), lambda i: (0, i)),
      out_shape=jax.ShapeDtypeStruct((NUM_CLASSES, np_), jnp.float32),
  )(aggp_t, xs_t, dinv, w1_t, b1_c, w2_t)


def _tc_pool(aggp_t, hws_t, dinv, batch_p, b2_r, bn=1792):
  np_ = hws_t.shape[1]
  grid = np_ // bn
  c = NUM_CLASSES

  def body(aggp_ref, hwst_ref, dinv_ref, batch_ref, b2_ref, out_ref, acc_ref,
           cnt_ref):
    i = pl.program_id(0)

    @pl.when(i == 0)
    def _():
      acc_ref[...] = jnp.zeros_like(acc_ref)
      cnt_ref[...] = jnp.zeros_like(cnt_ref)

    agg2 = dinv_ref[...] * (aggp_ref[0] + aggp_ref[1] + hwst_ref[...])
    m = (lax.broadcasted_iota(jnp.int32, (G, bn), 0) == batch_ref[...]
         ).astype(jnp.float32)
    acc_ref[...] += lax.dot_general(
        m, agg2, (((1,), (1,)), ((), ())), preferred_element_type=jnp.float32)
    cnt_ref[...] += jnp.sum(m, axis=1, keepdims=True)

    @pl.when(i == grid - 1)
    def _():
      pooled = acc_ref[...] / jnp.maximum(cnt_ref[...], 1.0) + b2_ref[...]
      mx = jnp.max(pooled, axis=1, keepdims=True)
      ex = jnp.exp(pooled - mx)
      out_ref[...] = (pooled - mx) - jnp.log(
          jnp.sum(ex, axis=1, keepdims=True))

  return pl.pallas_call(
      body,
      grid=(grid,),
      in_specs=[
          pl.BlockSpec((NC, c, bn), lambda i: (0, 0, i)),
          pl.BlockSpec((c, bn), lambda i: (0, i)),
          pl.BlockSpec((1, bn), lambda i: (0, i)),
          pl.BlockSpec((1, bn), lambda i: (0, i)),
          pl.BlockSpec((1, c), lambda i: (0, 0)),
      ],
      out_specs=pl.BlockSpec((G, c), lambda i: (0, 0)),
      out_shape=jax.ShapeDtypeStruct((G, c), jnp.float32),
      scratch_shapes=[
          pltpu.VMEM((G, c), jnp.float32),
          pltpu.VMEM((G, 1), jnp.float32),
      ],
  )(aggp_t, hws_t, dinv, batch_p, b2_r)


def kernel(x, edge_index, batch, W1, b1, W2, b2):
  n, f = x.shape
  e = edge_index.shape[1]

  np_ = -(-(n + 1) // 256) * 256
  rows_per_tile = -(-e // (NW * 128))
  rows_per_tile = -(-rows_per_tile // CH) * CH
  rows0 = (2 * rows_per_tile * 2 // 3) // CH * CH
  rows1 = 2 * rows_per_tile - rows0
  ep = NS * 128 * (rows0 + rows1)

  pad = jnp.full((ep - e,), n, jnp.int32)
  src2d = jnp.concatenate([edge_index[0], pad]).reshape(ep // 128, 128)
  dst2d = jnp.concatenate([edge_index[1], pad]).reshape(ep // 128, 128)

  z1 = jnp.zeros((np_,), jnp.float32)
  x_t = jnp.zeros((f, np_), jnp.float32).at[:, :n].set(x.T)
  batch_p = jnp.concatenate(
      [batch.astype(jnp.int32),
       jnp.full((np_ - n,), G, jnp.int32)]).reshape(1, np_)

  degp = _deg_count(dst2d, z1, np_, rows0, rows1)
  xs_t, dinv = _tc_scale(degp, x_t)
  agg1p = _edge_pass(src2d, dst2d, [xs_t[c] for c in range(f)], z1, np_, f,
                     rows0, rows1)
  hws_t = _tc_dense(agg1p, xs_t, dinv, W1.T, b1.reshape(-1, 1), W2.T)
  agg2p = _edge_pass(src2d, dst2d, [hws_t[c] for c in range(NUM_CLASSES)],
                     z1, np_, NUM_CLASSES, rows0, rows1)
  return _tc_pool(agg2p, hws_t, dinv, batch_p, b2.reshape(1, -1))


# speedup vs baseline: 1.3560x; 1.0294x over previous
import functools

import jax
import jax.numpy as jnp
from jax import lax
from jax.experimental import pallas as pl
from jax.experimental.pallas import tpu as pltpu
from jax.experimental.pallas import tpu_sc as plsc

NC = 2
NS = 16
NW = NC * NS
CH = 16
NUM_CLASSES = 6
HID = 160
G = 128


def _sc_mesh():
  return plsc.VectorSubcoreMesh(core_axis_name="c", subcore_axis_name="s")


def _deg_count(dst2d, zeros1, np_, rows0, rows1):
  rpt = np_ // NS

  @functools.partial(
      pl.kernel,
      out_type=jax.ShapeDtypeStruct((NC * np_,), jnp.float32),
      mesh=_sc_mesh(),
      scratch_types=[
          pltpu.VMEM_SHARED((np_,), jnp.float32),
          pltpu.VMEM((CH, 128), jnp.int32),
          pltpu.VMEM((128,), jnp.float32),
          pltpu.VMEM((np_ // NS,), jnp.float32),
          pltpu.SemaphoreType.DMA,
      ],
  )
  def k(dst_hbm, z_hbm, out_hbm, deg_sh, idx_v, ones_v, stage_v, sem):
    cid = lax.axis_index("c")
    sid = lax.axis_index("s")
    r0 = sid * rpt
    pltpu.sync_copy(z_hbm.at[pl.ds(r0, rpt)], stage_v)
    pltpu.sync_copy(stage_v, deg_sh.at[pl.ds(r0, rpt)])
    for i in range(8):
      ones_v[pl.ds(i * 16, 16)] = jnp.ones((16,), jnp.float32)
    plsc.subcore_barrier()
    rows_this = jnp.where(cid == 0, rows0, rows1)
    base = cid * NS * rows0 + sid * rows_this

    def blk(b, carry):
      pltpu.sync_copy(dst_hbm.at[pl.ds(base + b * CH, CH)], idx_v)
      descs = [
          pltpu.async_copy(ones_v, deg_sh.at[idx_v.at[j]], sem, add=True)
          for j in range(CH)
      ]
      for d in descs:
        d.wait()
      return carry

    lax.fori_loop(0, rows_this // CH, blk, 0)
    plsc.subcore_barrier()
    pltpu.sync_copy(deg_sh.at[pl.ds(r0, rpt)], stage_v)
    pltpu.sync_copy(stage_v, out_hbm.at[pl.ds(cid * np_ + r0, rpt)])

  return k(dst2d, zeros1).reshape(NC, np_)


def _edge_pass(src2d, dst2d, tab_cols, zeros1, np_, w, rows0, rows1):
  rpt = np_ // NS

  @functools.partial(
      pl.kernel,
      out_type=[jax.ShapeDtypeStruct((NC * np_,), jnp.float32)
                for _ in range(w)],
      mesh=_sc_mesh(),
      scratch_types=(
          [pltpu.VMEM_SHARED((np_,), jnp.float32) for _ in range(w)] +
          [pltpu.VMEM_SHARED((np_,), jnp.float32) for _ in range(w)] + [
              pltpu.VMEM((CH, 128), jnp.int32),
              pltpu.VMEM((CH, 128), jnp.int32),
              pltpu.VMEM((w, 128), jnp.float32),
              pltpu.VMEM((w, 128), jnp.float32),
              pltpu.VMEM((np_ // NS,), jnp.float32),
              pltpu.SemaphoreType.DMA,
              pltpu.SemaphoreType.DMA,
          ]),
  )
  def k(src_hbm, dst_hbm, *rest):
    tabs_hbm = rest[:w]
    z_hbm = rest[w]
    outs_hbm = rest[w + 1:2 * w + 1]
    tab_sh = rest[2 * w + 1:3 * w + 1]
    agg_sh = rest[3 * w + 1:4 * w + 1]
    sidx, didx, bufa, bufb, stage_v, semg, sems = rest[4 * w + 1:]
    cid = lax.axis_index("c")
    sid = lax.axis_index("s")
    r0 = sid * rpt
    pltpu.sync_copy(z_hbm.at[pl.ds(r0, rpt)], stage_v)
    for col in range(w):
      pltpu.sync_copy(stage_v, agg_sh[col].at[pl.ds(r0, rpt)])
    for col in range(w):
      pltpu.sync_copy(tabs_hbm[col].at[pl.ds(r0, rpt)], stage_v)
      pltpu.sync_copy(stage_v, tab_sh[col].at[pl.ds(r0, rpt)])
    plsc.subcore_barrier()
    rows_this = jnp.where(cid == 0, rows0, rows1)
    base = cid * NS * rows0 + sid * rows_this

    def blk(b, carry):
      pltpu.sync_copy(src_hbm.at[pl.ds(base + b * CH, CH)], sidx)
      pltpu.sync_copy(dst_hbm.at[pl.ds(base + b * CH, CH)], didx)

      def rowpair(p, c2):
        j0 = 2 * p
        j1 = j0 + 1
        ga = [
            pltpu.async_copy(tab_sh[col].at[sidx.at[j0]], bufa.at[col], semg)
            for col in range(w)
        ]
        gb = [
            pltpu.async_copy(tab_sh[col].at[sidx.at[j1]], bufb.at[col], semg)
            for col in range(w)
        ]
        for d in ga:
          d.wait()
        sa = [
            pltpu.async_copy(bufa.at[col], agg_sh[col].at[didx.at[j0]], sems,
                             add=True) for col in range(w)
        ]
        for d in gb:
          d.wait()
        sb = [
            pltpu.async_copy(bufb.at[col], agg_sh[col].at[didx.at[j1]], sems,
                             add=True) for col in range(w)
        ]
        for d in sa + sb:
          d.wait()
        return c2

      lax.fori_loop(0, CH // 2, rowpair, 0)
      return carry

    lax.fori_loop(0, rows_this // CH, blk, 0)
    plsc.subcore_barrier()
    for col in range(w):
      pltpu.sync_copy(agg_sh[col].at[pl.ds(r0, rpt)], stage_v)
      pltpu.sync_copy(stage_v, outs_hbm[col].at[pl.ds(cid * np_ + r0, rpt)])

  cols = k(src2d, dst2d, *tab_cols, zeros1)
  return jnp.concatenate(
      [c.reshape(NC, 1, np_) for c in cols], axis=1)


def _tc_scale(degp, x_t):
  np_ = degp.shape[1]
  f = x_t.shape[0]

  def body(degp_ref, xt_ref, xst_ref, dinv_ref):
    deg = degp_ref[0:1, :] + degp_ref[1:2, :] + 1.0
    dinv = lax.rsqrt(deg)
    dinv_ref[...] = dinv
    xst_ref[...] = xt_ref[...] * dinv

  return pl.pallas_call(
      body,
      out_shape=(
          jax.ShapeDtypeStruct((f, np_), jnp.float32),
          jax.ShapeDtypeStruct((1, np_), jnp.float32),
      ),
  )(degp, x_t)


def _tc_dense(aggp_t, xs_t, dinv, w1_t, b1_c, w2_t, bn=1792):
  np_ = xs_t.shape[1]
  grid = np_ // bn

  def body(aggp_ref, xst_ref, dinv_ref, w1t_ref, b1_ref, w2t_ref, out_ref):
    di = dinv_ref[...]
    agg = di * (aggp_ref[0] + aggp_ref[1] + xst_ref[...])
    h = jnp.dot(w1t_ref[...], agg, preferred_element_type=jnp.float32)
    h = jnp.maximum(h + b1_ref[...], 0.0)
    out_ref[...] = di * jnp.dot(
        w2t_ref[...], h, preferred_element_type=jnp.float32)

  return pl.pallas_call(
      body,
      grid=(grid,),
      in_specs=[
          pl.BlockSpec((NC, 4, bn), lambda i: (0, 0, i)),
          pl.BlockSpec((4, bn), lambda i: (0, i)),
          pl.BlockSpec((1, bn), lambda i: (0, i)),
          pl.BlockSpec((HID, 4), lambda i: (0, 0)),
          pl.BlockSpec((HID, 1), lambda i: (0, 0)),
          pl.BlockSpec((NUM_CLASSES, HID), lambda i: (0, 0)),
      ],
      out_specs=pl.BlockSpec((NUM_CLASSES, bn), lambda i: (0, i)),
      out_shape=jax.ShapeDtypeStruct((NUM_CLASSES, np_), jnp.float32),
  )(aggp_t, xs_t, dinv, w1_t, b1_c, w2_t)


def _tc_pool(aggp_t, hws_t, dinv, batch_p, b2_r, bn=1792):
  np_ = hws_t.shape[1]
  grid = np_ // bn
  c = NUM_CLASSES

  def body(aggp_ref, hwst_ref, dinv_ref, batch_ref, b2_ref, out_ref, acc_ref,
           cnt_ref):
    i = pl.program_id(0)

    @pl.when(i == 0)
    def _():
      acc_ref[...] = jnp.zeros_like(acc_ref)
      cnt_ref[...] = jnp.zeros_like(cnt_ref)

    agg2 = dinv_ref[...] * (aggp_ref[0] + aggp_ref[1] + hwst_ref[...])
    m = (lax.broadcasted_iota(jnp.int32, (G, bn), 0) == batch_ref[...]
         ).astype(jnp.float32)
    acc_ref[...] += lax.dot_general(
        m, agg2, (((1,), (1,)), ((), ())), preferred_element_type=jnp.float32)
    cnt_ref[...] += jnp.sum(m, axis=1, keepdims=True)

    @pl.when(i == grid - 1)
    def _():
      pooled = acc_ref[...] / jnp.maximum(cnt_ref[...], 1.0) + b2_ref[...]
      mx = jnp.max(pooled, axis=1, keepdims=True)
      ex = jnp.exp(pooled - mx)
      out_ref[...] = (pooled - mx) - jnp.log(
          jnp.sum(ex, axis=1, keepdims=True))

  return pl.pallas_call(
      body,
      grid=(grid,),
      in_specs=[
          pl.BlockSpec((NC, c, bn), lambda i: (0, 0, i)),
          pl.BlockSpec((c, bn), lambda i: (0, i)),
          pl.BlockSpec((1, bn), lambda i: (0, i)),
          pl.BlockSpec((1, bn), lambda i: (0, i)),
          pl.BlockSpec((1, c), lambda i: (0, 0)),
      ],
      out_specs=pl.BlockSpec((G, c), lambda i: (0, 0)),
      out_shape=jax.ShapeDtypeStruct((G, c), jnp.float32),
      scratch_shapes=[
          pltpu.VMEM((G, c), jnp.float32),
          pltpu.VMEM((G, 1), jnp.float32),
      ],
  )(aggp_t, hws_t, dinv, batch_p, b2_r)


def kernel(x, edge_index, batch, W1, b1, W2, b2):
  n, f = x.shape
  e = edge_index.shape[1]

  np_ = -(-(n + 1) // 256) * 256
  rows_per_tile = -(-e // (NW * 128))
  rows_per_tile = -(-rows_per_tile // CH) * CH
  rows0 = (2 * rows_per_tile * 9 // 13) // CH * CH
  rows1 = 2 * rows_per_tile - rows0
  ep = NS * 128 * (rows0 + rows1)

  pad = jnp.full((ep - e,), n, jnp.int32)
  src2d = jnp.concatenate([edge_index[0], pad]).reshape(ep // 128, 128)
  dst2d = jnp.concatenate([edge_index[1], pad]).reshape(ep // 128, 128)

  z1 = jnp.zeros((np_,), jnp.float32)
  x_t = jnp.zeros((f, np_), jnp.float32).at[:, :n].set(x.T)
  batch_p = jnp.concatenate(
      [batch.astype(jnp.int32),
       jnp.full((np_ - n,), G, jnp.int32)]).reshape(1, np_)

  degp = _deg_count(dst2d, z1, np_, rows0, rows1)
  xs_t, dinv = _tc_scale(degp, x_t)
  agg1p = _edge_pass(src2d, dst2d, [xs_t[c] for c in range(f)], z1, np_, f,
                     rows0, rows1)
  hws_t = _tc_dense(agg1p, xs_t, dinv, W1.T, b1.reshape(-1, 1), W2.T)
  agg2p = _edge_pass(src2d, dst2d, [hws_t[c] for c in range(NUM_CLASSES)],
                     z1, np_, NUM_CLASSES, rows0, rows1)
  return _tc_pool(agg2p, hws_t, dinv, batch_p, b2.reshape(1, -1))


# speedup vs baseline: 1.4087x; 1.0389x over previous
import functools

import jax
import jax.numpy as jnp
from jax import lax
from jax.experimental import pallas as pl
from jax.experimental.pallas import tpu as pltpu
from jax.experimental.pallas import tpu_sc as plsc

NC = 2
NS = 16
NW = NC * NS
CH = 16
NUM_CLASSES = 6
HID = 160
G = 128


def _sc_mesh():
  return plsc.VectorSubcoreMesh(core_axis_name="c", subcore_axis_name="s")


def _deg_count(dst2d, zeros1, np_, rows0, rows1):
  rpt = np_ // NS

  @functools.partial(
      pl.kernel,
      out_type=jax.ShapeDtypeStruct((NC * np_,), jnp.float32),
      mesh=_sc_mesh(),
      scratch_types=[
          pltpu.VMEM_SHARED((np_,), jnp.float32),
          pltpu.VMEM((CH, 128), jnp.int32),
          pltpu.VMEM((128,), jnp.float32),
          pltpu.VMEM((np_ // NS,), jnp.float32),
          pltpu.SemaphoreType.DMA,
      ],
  )
  def k(dst_hbm, z_hbm, out_hbm, deg_sh, idx_v, ones_v, stage_v, sem):
    cid = lax.axis_index("c")
    sid = lax.axis_index("s")
    r0 = sid * rpt
    pltpu.sync_copy(z_hbm.at[pl.ds(r0, rpt)], stage_v)
    pltpu.sync_copy(stage_v, deg_sh.at[pl.ds(r0, rpt)])
    for i in range(8):
      ones_v[pl.ds(i * 16, 16)] = jnp.ones((16,), jnp.float32)
    plsc.subcore_barrier()
    rows_this = jnp.where(cid == 0, rows0, rows1)
    base = cid * NS * rows0 + sid * rows_this

    def blk(b, carry):
      pltpu.sync_copy(dst_hbm.at[pl.ds(base + b * CH, CH)], idx_v)
      descs = [
          pltpu.async_copy(ones_v, deg_sh.at[idx_v.at[j]], sem, add=True)
          for j in range(CH)
      ]
      for d in descs:
        d.wait()
      return carry

    lax.fori_loop(0, rows_this // CH, blk, 0)
    plsc.subcore_barrier()
    pltpu.sync_copy(deg_sh.at[pl.ds(r0, rpt)], stage_v)
    pltpu.sync_copy(stage_v, out_hbm.at[pl.ds(cid * np_ + r0, rpt)])

  return k(dst2d, zeros1).reshape(NC, np_)


def _edge_pass(src2d, dst2d, tab_cols, zeros1, np_, w, rows0, rows1):
  rpt = np_ // NS

  @functools.partial(
      pl.kernel,
      out_type=[jax.ShapeDtypeStruct((NC * np_,), jnp.float32)
                for _ in range(w)],
      mesh=_sc_mesh(),
      scratch_types=(
          [pltpu.VMEM_SHARED((np_,), jnp.float32) for _ in range(w)] +
          [pltpu.VMEM_SHARED((np_,), jnp.float32) for _ in range(w)] + [
              pltpu.VMEM((CH, 128), jnp.int32),
              pltpu.VMEM((CH, 128), jnp.int32),
              pltpu.VMEM((w, 128), jnp.float32),
              pltpu.VMEM((w, 128), jnp.float32),
              pltpu.VMEM((np_ // NS,), jnp.float32),
              pltpu.SemaphoreType.DMA,
              pltpu.SemaphoreType.DMA,
          ]),
  )
  def k(src_hbm, dst_hbm, *rest):
    tabs_hbm = rest[:w]
    z_hbm = rest[w]
    outs_hbm = rest[w + 1:2 * w + 1]
    tab_sh = rest[2 * w + 1:3 * w + 1]
    agg_sh = rest[3 * w + 1:4 * w + 1]
    sidx, didx, bufa, bufb, stage_v, semg, sems = rest[4 * w + 1:]
    cid = lax.axis_index("c")
    sid = lax.axis_index("s")
    r0 = sid * rpt
    pltpu.sync_copy(z_hbm.at[pl.ds(r0, rpt)], stage_v)
    for col in range(w):
      pltpu.sync_copy(stage_v, agg_sh[col].at[pl.ds(r0, rpt)])
    for col in range(w):
      pltpu.sync_copy(tabs_hbm[col].at[pl.ds(r0, rpt)], stage_v)
      pltpu.sync_copy(stage_v, tab_sh[col].at[pl.ds(r0, rpt)])
    plsc.subcore_barrier()
    rows_this = jnp.where(cid == 0, rows0, rows1)
    base = cid * NS * rows0 + sid * rows_this

    def blk(b, carry):
      pltpu.sync_copy(src_hbm.at[pl.ds(base + b * CH, CH)], sidx)
      pltpu.sync_copy(dst_hbm.at[pl.ds(base + b * CH, CH)], didx)

      def rowpair(p, c2):
        j0 = 2 * p
        j1 = j0 + 1
        ga = [
            pltpu.async_copy(tab_sh[col].at[sidx.at[j0]], bufa.at[col], semg)
            for col in range(w)
        ]
        gb = [
            pltpu.async_copy(tab_sh[col].at[sidx.at[j1]], bufb.at[col], semg)
            for col in range(w)
        ]
        for d in ga:
          d.wait()
        sa = [
            pltpu.async_copy(bufa.at[col], agg_sh[col].at[didx.at[j0]], sems,
                             add=True) for col in range(w)
        ]
        for d in gb:
          d.wait()
        sb = [
            pltpu.async_copy(bufb.at[col], agg_sh[col].at[didx.at[j1]], sems,
                             add=True) for col in range(w)
        ]
        for d in sa + sb:
          d.wait()
        return c2

      lax.fori_loop(0, CH // 2, rowpair, 0)
      return carry

    lax.fori_loop(0, rows_this // CH, blk, 0)
    plsc.subcore_barrier()
    for col in range(w):
      pltpu.sync_copy(agg_sh[col].at[pl.ds(r0, rpt)], stage_v)
      pltpu.sync_copy(stage_v, outs_hbm[col].at[pl.ds(cid * np_ + r0, rpt)])

  cols = k(src2d, dst2d, *tab_cols, zeros1)
  return jnp.concatenate(
      [c.reshape(NC, 1, np_) for c in cols], axis=1)


def _tc_scale(degp, x_t):
  np_ = degp.shape[1]
  f = x_t.shape[0]

  def body(degp_ref, xt_ref, xst_ref, dinv_ref):
    deg = degp_ref[0:1, :] + degp_ref[1:2, :] + 1.0
    dinv = lax.rsqrt(deg)
    dinv_ref[...] = dinv
    xst_ref[...] = xt_ref[...] * dinv

  return pl.pallas_call(
      body,
      out_shape=(
          jax.ShapeDtypeStruct((f, np_), jnp.float32),
          jax.ShapeDtypeStruct((1, np_), jnp.float32),
      ),
  )(degp, x_t)


def _tc_dense(aggp_t, xs_t, dinv, w1_t, b1_c, w2_t, bn=1792):
  np_ = xs_t.shape[1]
  grid = np_ // bn

  def body(aggp_ref, xst_ref, dinv_ref, w1t_ref, b1_ref, w2t_ref, out_ref):
    di = dinv_ref[...]
    agg = di * (aggp_ref[0] + aggp_ref[1] + xst_ref[...])
    h = jnp.dot(w1t_ref[...], agg, preferred_element_type=jnp.float32)
    h = jnp.maximum(h + b1_ref[...], 0.0)
    out_ref[...] = di * jnp.dot(
        w2t_ref[...], h, preferred_element_type=jnp.float32)

  return pl.pallas_call(
      body,
      grid=(grid,),
      in_specs=[
          pl.BlockSpec((NC, 4, bn), lambda i: (0, 0, i)),
          pl.BlockSpec((4, bn), lambda i: (0, i)),
          pl.BlockSpec((1, bn), lambda i: (0, i)),
          pl.BlockSpec((HID, 4), lambda i: (0, 0)),
          pl.BlockSpec((HID, 1), lambda i: (0, 0)),
          pl.BlockSpec((NUM_CLASSES, HID), lambda i: (0, 0)),
      ],
      out_specs=pl.BlockSpec((NUM_CLASSES, bn), lambda i: (0, i)),
      out_shape=jax.ShapeDtypeStruct((NUM_CLASSES, np_), jnp.float32),
  )(aggp_t, xs_t, dinv, w1_t, b1_c, w2_t)


def _tc_pool(aggp_t, hws_t, dinv, batch_p, b2_r, bn=1792):
  np_ = hws_t.shape[1]
  grid = np_ // bn
  c = NUM_CLASSES

  def body(aggp_ref, hwst_ref, dinv_ref, batch_ref, b2_ref, out_ref, acc_ref,
           cnt_ref):
    i = pl.program_id(0)

    @pl.when(i == 0)
    def _():
      acc_ref[...] = jnp.zeros_like(acc_ref)
      cnt_ref[...] = jnp.zeros_like(cnt_ref)

    agg2 = dinv_ref[...] * (aggp_ref[0] + aggp_ref[1] + hwst_ref[...])
    m = (lax.broadcasted_iota(jnp.int32, (G, bn), 0) == batch_ref[...]
         ).astype(jnp.float32)
    acc_ref[...] += lax.dot_general(
        m, agg2, (((1,), (1,)), ((), ())), preferred_element_type=jnp.float32)
    cnt_ref[...] += jnp.sum(m, axis=1, keepdims=True)

    @pl.when(i == grid - 1)
    def _():
      pooled = acc_ref[...] / jnp.maximum(cnt_ref[...], 1.0) + b2_ref[...]
      mx = jnp.max(pooled, axis=1, keepdims=True)
      ex = jnp.exp(pooled - mx)
      out_ref[...] = (pooled - mx) - jnp.log(
          jnp.sum(ex, axis=1, keepdims=True))

  return pl.pallas_call(
      body,
      grid=(grid,),
      in_specs=[
          pl.BlockSpec((NC, c, bn), lambda i: (0, 0, i)),
          pl.BlockSpec((c, bn), lambda i: (0, i)),
          pl.BlockSpec((1, bn), lambda i: (0, i)),
          pl.BlockSpec((1, bn), lambda i: (0, i)),
          pl.BlockSpec((1, c), lambda i: (0, 0)),
      ],
      out_specs=pl.BlockSpec((G, c), lambda i: (0, 0)),
      out_shape=jax.ShapeDtypeStruct((G, c), jnp.float32),
      scratch_shapes=[
          pltpu.VMEM((G, c), jnp.float32),
          pltpu.VMEM((G, 1), jnp.float32),
      ],
  )(aggp_t, hws_t, dinv, batch_p, b2_r)


def kernel(x, edge_index, batch, W1, b1, W2, b2):
  n, f = x.shape
  e = edge_index.shape[1]

  np_ = -(-(n + 1) // 256) * 256
  rows_per_tile = -(-e // (NW * 128))
  rows_per_tile = -(-rows_per_tile // CH) * CH
  rows0 = (2 * rows_per_tile * 19 // 26) // CH * CH
  rows1 = 2 * rows_per_tile - rows0
  ep = NS * 128 * (rows0 + rows1)

  pad = jnp.full((ep - e,), n, jnp.int32)
  src2d = jnp.concatenate([edge_index[0], pad]).reshape(ep // 128, 128)
  dst2d = jnp.concatenate([edge_index[1], pad]).reshape(ep // 128, 128)

  z1 = jnp.zeros((np_,), jnp.float32)
  x_t = jnp.zeros((f, np_), jnp.float32).at[:, :n].set(x.T)
  batch_p = jnp.concatenate(
      [batch.astype(jnp.int32),
       jnp.full((np_ - n,), G, jnp.int32)]).reshape(1, np_)

  degp = _deg_count(dst2d, z1, np_, rows0, rows1)
  xs_t, dinv = _tc_scale(degp, x_t)
  agg1p = _edge_pass(src2d, dst2d, [xs_t[c] for c in range(f)], z1, np_, f,
                     rows0, rows1)
  hws_t = _tc_dense(agg1p, xs_t, dinv, W1.T, b1.reshape(-1, 1), W2.T)
  agg2p = _edge_pass(src2d, dst2d, [hws_t[c] for c in range(NUM_CLASSES)],
                     z1, np_, NUM_CLASSES, rows0, rows1)
  return _tc_pool(agg2p, hws_t, dinv, batch_p, b2.reshape(1, -1))
